# baseline (speedup 1.0000x reference)
import jax
import jax.numpy as jnp
from jax import lax
from jax.experimental import pallas as pl
from jax.experimental.pallas import tpu as pltpu
from jax.experimental.pallas import tpu_sc as plsc

_NC = 2
_NS = 16
_NW = _NC * _NS
_LANES = 16
_NBUF = 4


def _make_body(batch, seq, embed, bc):
  n_groups = bc // _LANES

  def body(x_hbm, table_hbm, out_hbm, xbuf, idx_bufs, row_bufs, sems):
    wid = lax.axis_index("s") * _NC + lax.axis_index("c")
    b0 = wid * bc
    pltpu.sync_copy(x_hbm.at[pl.ds(b0 * seq, bc * seq)], xbuf)

    lane = lax.iota(jnp.int32, _LANES)

    def build_idx(s, idx_buf):
      for j in range(n_groups):
        pos = (j * _LANES) * seq + lane * seq + s
        idx_buf[pl.ds(j * _LANES, _LANES)] = plsc.load_gather(xbuf, [pos])

    def start(s, k):
      build_idx(s, idx_bufs[k])
      pltpu.async_copy(table_hbm.at[idx_bufs[k]], row_bufs[k], sems[k])

    for k in range(_NBUF):
      start(k, k)

    def step(g, carry):
      for k in range(_NBUF):
        s = g * _NBUF + k
        pltpu.make_async_copy(
            table_hbm.at[idx_bufs[k]], row_bufs[k], sems[k]).wait()
        pltpu.sync_copy(row_bufs[k], out_hbm.at[pl.ds(s * batch + b0, bc)])
        nxt = s + _NBUF

        @pl.when(nxt < seq)
        def _():
          start(nxt, k)
      return carry

    lax.fori_loop(0, seq // _NBUF, step, None)

  return body


@jax.jit
def kernel(x, table):
  batch, seq = x.shape
  _, embed = table.shape
  bc = batch // _NW
  x_flat = x.reshape(-1)

  mesh = plsc.VectorSubcoreMesh(core_axis_name="c", subcore_axis_name="s")
  out = pl.kernel(
      _make_body(batch, seq, embed, bc),
      out_type=jax.ShapeDtypeStruct((seq * batch, embed), jnp.float32),
      mesh=mesh,
      compiler_params=pltpu.CompilerParams(
          needs_layout_passes=False, use_tc_tiling_on_sc=False),
      scratch_types=[
          pltpu.VMEM((bc * seq,), jnp.int32),
          [pltpu.VMEM((bc,), jnp.int32) for _ in range(_NBUF)],
          [pltpu.VMEM((bc, embed), jnp.float32) for _ in range(_NBUF)],
          [pltpu.SemaphoreType.DMA for _ in range(_NBUF)],
      ],
  )(x_flat, table)
  return out.reshape(seq, batch, embed)
